# baseline (speedup 1.0000x reference)
import jax
import jax.numpy as jnp
from jax.experimental import pallas as pl

_VALUE = 0.075
_ROWS = 128
_VOCAB = 100000
_BLOCK_ROWS = 8

_noise_consts = None


def _get_noise_consts():
    global _noise_consts
    if _noise_consts is None:
        nkey = jax.random.key(1)
        noise = jax.random.normal(nkey, (_ROWS, _VOCAB), dtype=jnp.float32)
        noise = noise * _VALUE
        noise_max = jnp.max(noise, axis=1, keepdims=True)
        _noise_consts = (noise, noise_max)
    return _noise_consts


def _body(x_ref, n_ref, nmax_ref, o_ref):
    xb = x_ref[...]
    m1 = jnp.max(xb, axis=1, keepdims=True)
    e1 = jnp.exp(xb - m1)
    s1 = jnp.sum(e1, axis=1, keepdims=True)
    conf = e1 / s1
    cmax = jnp.max(conf, axis=1, keepdims=True)
    ids = jax.lax.broadcasted_iota(jnp.int32, conf.shape, 1)
    top1 = jnp.min(jnp.where(conf == cmax, ids, _VOCAB), axis=1, keepdims=True)
    y = jnp.where(ids == top1, cmax - nmax_ref[...], conf + n_ref[...])
    m2 = jnp.max(y, axis=1, keepdims=True)
    e2 = jnp.exp(y - m2)
    s2 = jnp.sum(e2, axis=1, keepdims=True)
    o_ref[...] = e2 / s2


def kernel(x):
    noise, noise_max = _get_noise_consts()
    grid = (_ROWS // _BLOCK_ROWS,)
    return pl.pallas_call(
        _body,
        grid=grid,
        in_specs=[
            pl.BlockSpec((_BLOCK_ROWS, _VOCAB), lambda i: (i, 0)),
            pl.BlockSpec((_BLOCK_ROWS, _VOCAB), lambda i: (i, 0)),
            pl.BlockSpec((_BLOCK_ROWS, 1), lambda i: (i, 0)),
        ],
        out_specs=pl.BlockSpec((_BLOCK_ROWS, _VOCAB), lambda i: (i, 0)),
        out_shape=jax.ShapeDtypeStruct((_ROWS, _VOCAB), jnp.float32),
    )(x, noise, noise_max)


# speedup vs baseline: 1.0461x; 1.0461x over previous
import jax
import jax.numpy as jnp
from jax.experimental import pallas as pl

_VALUE = 0.075
_ROWS = 128
_VOCAB = 100000
_BLOCK_ROWS = 8

_noise_consts = None


def _get_noise_consts():
    global _noise_consts
    if _noise_consts is None:
        nkey = jax.random.key(1)
        noise = jax.random.normal(nkey, (_ROWS, _VOCAB), dtype=jnp.float32)
        noise = noise * _VALUE
        noise_max = jnp.max(noise, axis=1, keepdims=True)
        _noise_consts = (noise, noise_max)
    return _noise_consts


def _body(x_ref, n_ref, nmax_ref, o_ref):
    xb = x_ref[...]
    m1 = jnp.max(xb, axis=1, keepdims=True)
    e1 = jnp.exp(xb - m1)
    s1 = jnp.sum(e1, axis=1, keepdims=True)
    inv1 = 1.0 / s1
    ids = jax.lax.broadcasted_iota(jnp.int32, xb.shape, 1)
    top1 = jnp.min(jnp.where(e1 == 1.0, ids, _VOCAB), axis=1, keepdims=True)
    t = jnp.exp(e1 * inv1 + n_ref[...])
    ttop = jnp.exp(inv1 - nmax_ref[...])
    t = jnp.where(ids == top1, ttop, t)
    s2 = jnp.sum(t, axis=1, keepdims=True)
    o_ref[...] = t * (1.0 / s2)


def kernel(x):
    noise, noise_max = _get_noise_consts()
    grid = (_ROWS // _BLOCK_ROWS,)
    return pl.pallas_call(
        _body,
        grid=grid,
        in_specs=[
            pl.BlockSpec((_BLOCK_ROWS, _VOCAB), lambda i: (i, 0)),
            pl.BlockSpec((_BLOCK_ROWS, _VOCAB), lambda i: (i, 0)),
            pl.BlockSpec((_BLOCK_ROWS, 1), lambda i: (i, 0)),
        ],
        out_specs=pl.BlockSpec((_BLOCK_ROWS, _VOCAB), lambda i: (i, 0)),
        out_shape=jax.ShapeDtypeStruct((_ROWS, _VOCAB), jnp.float32),
    )(x, noise, noise_max)
